# baseline (speedup 1.0000x reference)
import functools

import jax
import jax.numpy as jnp
from jax import lax
from jax.experimental import pallas as pl
from jax.experimental.pallas import tpu as pltpu
from jax.experimental.pallas import tpu_sc as plsc

_L = 16
_B = 4
_N = 4096
_NQ = 1024
_QV = _NQ // _L
_SB = 4


def _sc_chamfer_body(x1x, x1y, x1z, x2x, x2y, x2z, d1, d2,
                     qx, qy, qz, qs, rx, ry, rz, rs, ov):
    nc = 2
    wid = lax.axis_index("s") * nc + lax.axis_index("c")
    b = wid % _B
    dr = (wid // _B) % 2
    c = wid // (_B * 2)
    qo = b * _N + c * _NQ
    ro = b * _N

    @pl.when(dr == 0)
    def _():
        pltpu.sync_copy(x1x.at[pl.ds(qo, _NQ)], qx)
        pltpu.sync_copy(x1y.at[pl.ds(qo, _NQ)], qy)
        pltpu.sync_copy(x1z.at[pl.ds(qo, _NQ)], qz)
        pltpu.sync_copy(x2x.at[pl.ds(ro, _N)], rx)
        pltpu.sync_copy(x2y.at[pl.ds(ro, _N)], ry)
        pltpu.sync_copy(x2z.at[pl.ds(ro, _N)], rz)

    @pl.when(dr == 1)
    def _():
        pltpu.sync_copy(x2x.at[pl.ds(qo, _NQ)], qx)
        pltpu.sync_copy(x2y.at[pl.ds(qo, _NQ)], qy)
        pltpu.sync_copy(x2z.at[pl.ds(qo, _NQ)], qz)
        pltpu.sync_copy(x1x.at[pl.ds(ro, _N)], rx)
        pltpu.sync_copy(x1y.at[pl.ds(ro, _N)], ry)
        pltpu.sync_copy(x1z.at[pl.ds(ro, _N)], rz)

    def _bf16r(v):
        y = v * jnp.float32(65537.0)
        return y - (y - v)

    def _rnorm(i, _):
        s = pl.ds(i * _L, _L)
        vx, vy, vz = rx[s], ry[s], rz[s]
        rs[s] = vx * vx + vy * vy + vz * vz
        rx[s] = _bf16r(vx)
        ry[s] = _bf16r(vy)
        rz[s] = _bf16r(vz)
        return 0

    lax.fori_loop(0, _N // _L, _rnorm, 0)

    def _qnorm(i, _):
        s = pl.ds(i * _L, _L)
        vx, vy, vz = qx[s], qy[s], qz[s]
        qs[s] = vx * vx + vy * vy + vz * vz
        qx[s] = -2.0 * _bf16r(vx)
        qy[s] = -2.0 * _bf16r(vy)
        qz[s] = -2.0 * _bf16r(vz)
        return 0

    lax.fori_loop(0, _QV, _qnorm, 0)

    inf16 = jnp.full((_L,), jnp.inf, jnp.float32)

    def _superblock(sb, _):
        base = sb * (_SB * _L)
        mqx = [qx[pl.ds(base + t * _L, _L)] for t in range(_SB)]
        mqy = [qy[pl.ds(base + t * _L, _L)] for t in range(_SB)]
        mqz = [qz[pl.ds(base + t * _L, _L)] for t in range(_SB)]
        nqs = [qs[pl.ds(base + t * _L, _L)] for t in range(_SB)]

        def _jstep(jc, accs):
            s = pl.ds(jc * _L, _L)
            rx16, ry16, rz16, rs16 = rx[s], ry[s], rz[s], rs[s]
            accs = list(accs)
            for jj in range(_L):
                rxj, ryj, rzj, rsj = rx16[jj], ry16[jj], rz16[jj], rs16[jj]
                for t in range(_SB):
                    d = (nqs[t] + rsj) + mqx[t] * rxj + mqy[t] * ryj \
                        + mqz[t] * rzj
                    accs[t] = jnp.minimum(accs[t], d)
            return tuple(accs)

        accs = lax.fori_loop(0, _N // _L, _jstep, (inf16,) * _SB)
        for t in range(_SB):
            ov[pl.ds(base + t * _L, _L)] = accs[t]
        return 0

    lax.fori_loop(0, _QV // _SB, _superblock, 0)

    @pl.when(dr == 0)
    def _():
        pltpu.sync_copy(ov, d1.at[pl.ds(qo, _NQ)])

    @pl.when(dr == 1)
    def _():
        pltpu.sync_copy(ov, d2.at[pl.ds(qo, _NQ)])


def _sc_chamfer(x1x, x1y, x1z, x2x, x2y, x2z):
    f32 = jnp.float32
    mesh = plsc.VectorSubcoreMesh(core_axis_name="c", subcore_axis_name="s")
    k = functools.partial(
        pl.kernel,
        mesh=mesh,
        out_type=[
            jax.ShapeDtypeStruct((_B * _N,), f32),
            jax.ShapeDtypeStruct((_B * _N,), f32),
        ],
        scratch_types=[
            pltpu.VMEM((_NQ,), f32),
            pltpu.VMEM((_NQ,), f32),
            pltpu.VMEM((_NQ,), f32),
            pltpu.VMEM((_NQ,), f32),
            pltpu.VMEM((_N,), f32),
            pltpu.VMEM((_N,), f32),
            pltpu.VMEM((_N,), f32),
            pltpu.VMEM((_N,), f32),
            pltpu.VMEM((_NQ,), f32),
        ],
    )(_sc_chamfer_body)
    return k(x1x, x1y, x1z, x2x, x2y, x2z)


@jax.jit
def kernel(xyz1, xyz2):
    B, N, _ = xyz1.shape
    M = xyz2.shape[1]
    x1x, x1y, x1z = [xyz1[:, :, k].reshape(-1) for k in range(3)]
    x2x, x2y, x2z = [xyz2[:, :, k].reshape(-1) for k in range(3)]
    d1, d2 = _sc_chamfer(x1x, x1y, x1z, x2x, x2y, x2z)
    return d1.reshape(B, N), d2.reshape(B, M)


# speedup vs baseline: 36.9312x; 36.9312x over previous
import functools

import jax
import jax.numpy as jnp
from jax.experimental import pallas as pl
from jax.experimental.pallas import tpu as pltpu

_BN = 2048


def _cd_body(x1a_ref, x2a_ref, d1_ref, d2_ref):
    nb = pl.program_id(1)
    x1a = x1a_ref[0]
    x2a = x2a_ref[0]
    d = jax.lax.dot_general(
        x1a, x2a, dimension_numbers=(((1,), (0,)), ((), ())),
        preferred_element_type=jnp.float32)
    d1_ref[0] = jnp.min(d, axis=1, keepdims=True)
    part = jnp.min(d, axis=0, keepdims=True)

    @pl.when(nb == 0)
    def _():
        d2_ref[0] = part

    @pl.when(nb > 0)
    def _():
        d2_ref[0] = jnp.minimum(d2_ref[0], part)


def _bf16_split3(v):
    h1 = v.astype(jnp.bfloat16).astype(jnp.float32)
    r1 = v - h1
    h2 = r1.astype(jnp.bfloat16).astype(jnp.float32)
    r2 = r1 - h2
    h3 = r2.astype(jnp.bfloat16).astype(jnp.float32)
    return h1, h2, h3


@jax.jit
def kernel(xyz1, xyz2):
    B, N, _ = xyz1.shape
    M = xyz2.shape[1]
    x1s = jnp.sum(xyz1 * xyz1, axis=-1, keepdims=True)
    x2s = jnp.sum(xyz2 * xyz2, axis=-1, keepdims=True)
    a1, a2, a3 = _bf16_split3(x1s)
    b1, b2, b3 = _bf16_split3(x2s)
    one1 = jnp.ones((B, N, 1), jnp.float32)
    one2 = jnp.ones((B, M, 1), jnp.float32)
    x1a = jnp.concatenate(
        [-2.0 * xyz1, a1, a2, a3, one1, one1, one1], axis=-1)
    x2a = jnp.transpose(
        jnp.concatenate([xyz2, one2, one2, one2, b1, b2, b3], axis=-1),
        (0, 2, 1))
    grid = (B, N // _BN)
    d1, d2 = pl.pallas_call(
        _cd_body,
        grid=grid,
        in_specs=[
            pl.BlockSpec((1, _BN, 9), lambda b, i: (b, i, 0)),
            pl.BlockSpec((1, 9, M), lambda b, i: (b, 0, 0)),
        ],
        out_specs=[
            pl.BlockSpec((1, _BN, 1), lambda b, i: (b, i, 0)),
            pl.BlockSpec((1, 1, M), lambda b, i: (b, 0, 0)),
        ],
        out_shape=[
            jax.ShapeDtypeStruct((B, N, 1), jnp.float32),
            jax.ShapeDtypeStruct((B, 1, M), jnp.float32),
        ],
        compiler_params=pltpu.CompilerParams(
            dimension_semantics=("parallel", "arbitrary")),
    )(x1a, x2a)
    return d1.reshape(B, N), d2.reshape(B, M)
